# baseline (speedup 1.0000x reference)
import functools

import jax
import jax.numpy as jnp
from jax import lax
from jax.experimental import pallas as pl
from jax.experimental.pallas import tpu as pltpu

_N = 1024
_C = 81
_H = 28
_HH = _H * _H
_BN = 8
_GRID = _N // _BN


def _loss_body(pm_blk, cl, lg, td, d2, tm, out, acc):
    g = pl.program_id(0)

    @pl.when(g == 0)
    def _init():
        for k in range(5):
            acc[k] = 0.0

    cls = cl[...]
    keep = (cls != -1).astype(jnp.float32)
    pos = (cls > 0).astype(jnp.float32)
    safe = jnp.maximum(cls, 0)

    logits = lg[...]
    m = jnp.max(logits, axis=1, keepdims=True)
    lse = m + jnp.log(jnp.sum(jnp.exp(logits - m), axis=1, keepdims=True))
    iota = lax.broadcasted_iota(jnp.int32, logits.shape, 1)
    picked = jnp.sum(jnp.where(iota == safe, logits, 0.0), axis=1,
                     keepdims=True)
    acc[0] += jnp.sum(keep * (lse - picked))
    acc[1] += jnp.sum(keep)

    dv = d2[...]
    tdt = td[...]
    iota2 = lax.broadcasted_iota(jnp.int32, dv.shape, 1)
    selm = (iota2 // 4) == safe
    diff = jnp.abs(tdt - dv)
    lt = (diff < 1.0).astype(jnp.float32)
    sl1 = lt * 0.5 * diff * diff + (1.0 - lt) * (diff - 0.5)
    acc[2] += jnp.sum(jnp.where(selm, pos * sl1, 0.0))
    acc[3] += jnp.sum(pos)

    b = pm_blk[...]
    s = 0.0
    for i in range(_BN):
        mi = b[i].reshape(_HH, _C)
        cid = cls[i, 0]
        cid_safe = jnp.maximum(cid, 0)
        onehot = (lax.broadcasted_iota(jnp.int32, (1, _C), 1)
                  == cid_safe).astype(jnp.float32)
        ypn = lax.dot_general(mi, onehot, (((1,), (1,)), ((), ())),
                              preferred_element_type=jnp.float32)
        ypr = ypn.reshape(1, _HH)
        tmn = tm[pl.ds(i, 1), :]
        posn = (cid > 0).astype(jnp.float32)
        sce = (jnp.maximum(ypr, 0.0) - ypr * tmn
               + jnp.log1p(jnp.exp(-jnp.abs(ypr))))
        s = s + posn * jnp.sum(sce)
    acc[4] += s

    @pl.when(g == pl.num_programs(0) - 1)
    def _finalize():
        total = (acc[0] / acc[1]
                 + acc[2] / (acc[3] * 4.0)
                 + acc[4] / (acc[3] * float(_HH)))
        out[...] = jnp.full((1, 1), total, dtype=jnp.float32)


def _fused_loss(pm, cls2, logits, tdt, d2, tm2):
    return pl.pallas_call(
        _loss_body,
        grid=(_GRID,),
        in_specs=[
            pl.BlockSpec((_BN, _H, _H, _C), lambda g: (g, 0, 0, 0)),
            pl.BlockSpec((_BN, 1), lambda g: (g, 0)),
            pl.BlockSpec((_BN, _C), lambda g: (g, 0)),
            pl.BlockSpec((_BN, 4 * _C), lambda g: (g, 0)),
            pl.BlockSpec((_BN, 4 * _C), lambda g: (g, 0)),
            pl.BlockSpec((_BN, _HH), lambda g: (g, 0)),
        ],
        out_specs=pl.BlockSpec((1, 1), lambda g: (0, 0)),
        out_shape=jax.ShapeDtypeStruct((1, 1), jnp.float32),
        scratch_shapes=[pltpu.SMEM((8,), jnp.float32)],
    )(pm, cls2, logits, tdt, d2, tm2)


def kernel(target_deltas, mrcnn_deltas, mrcnn_class_logits, target_masks,
           pred_masks, target_class_ids):
    cls = target_class_ids.astype(jnp.int32)

    tm2 = target_masks.reshape(_N, _HH)
    d2 = mrcnn_deltas.reshape(_N, _C * 4)
    tdt = jnp.tile(target_deltas, (1, _C))

    out = _fused_loss(pred_masks, cls[:, None], mrcnn_class_logits, tdt, d2,
                      tm2)
    return out[0, 0]


# speedup vs baseline: 1.7086x; 1.7086x over previous
import functools

import jax
import jax.numpy as jnp
from jax import lax
from jax.experimental import pallas as pl
from jax.experimental.pallas import tpu as pltpu

_N = 1024
_C = 81
_H = 28
_HH = _H * _H
_BN = 16
_NB = _N // _BN
_WB = 4
_BR = _BN * _H
_SEG = _H * 8


def _loss_body(pm_blk, cl, lg, td, d2, tm, out, acc):
    nb = pl.program_id(0)
    wb = pl.program_id(1)

    @pl.when((nb == 0) & (wb == 0))
    def _init():
        for k in range(5):
            acc[k] = 0.0

    cls = cl[...]
    safe = jnp.maximum(cls, 0)

    @pl.when(wb == 0)
    def _class_bbox():
        keep = (cls != -1).astype(jnp.float32)
        pos = (cls > 0).astype(jnp.float32)

        logits = lg[...]
        m = jnp.max(logits, axis=1, keepdims=True)
        lse = m + jnp.log(jnp.sum(jnp.exp(logits - m), axis=1,
                                  keepdims=True))
        iota = lax.broadcasted_iota(jnp.int32, logits.shape, 1)
        picked = jnp.sum(jnp.where(iota == safe, logits, 0.0), axis=1,
                         keepdims=True)
        acc[0] += jnp.sum(keep * (lse - picked))
        acc[1] += jnp.sum(keep)

        dv = d2[...]
        tdt = td[...]
        iota2 = lax.broadcasted_iota(jnp.int32, dv.shape, 1)
        selm = (iota2 // 4) == safe
        diff = jnp.abs(tdt - dv)
        lt = (diff < 1.0).astype(jnp.float32)
        sl1 = lt * 0.5 * diff * diff + (1.0 - lt) * (diff - 0.5)
        acc[2] += jnp.sum(jnp.where(selm, pos * sl1, 0.0))
        acc[3] += jnp.sum(pos)

    b = pm_blk[...].reshape(_BR * 8, _C)
    iota_c = lax.broadcasted_iota(jnp.int32, (_BN, _C), 1)
    onehot = (iota_c == safe).astype(jnp.float32)
    sel = lax.dot_general(onehot, b, (((1,), (1,)), ((), ())),
                          preferred_element_type=jnp.float32)

    valid = (lax.broadcasted_iota(jnp.int32, (1, _SEG), 1) % 8
             + 8 * wb) < _H
    pos_all = (cls > 0).astype(jnp.float32)
    s = 0.0
    for i in range(_BN):
        ypr = lax.slice(sel, (i, i * _SEG), (i + 1, (i + 1) * _SEG))
        tmn = tm[pl.ds(i, 1), 0, 0, :]
        posn = lax.slice(pos_all, (i, 0), (i + 1, 1))
        sce = (jnp.maximum(ypr, 0.0) - ypr * tmn
               + jnp.log1p(jnp.exp(-jnp.abs(ypr))))
        s = s + jnp.sum(posn * jnp.where(valid, sce, 0.0))
    acc[4] += s

    @pl.when((nb == pl.num_programs(0) - 1) & (wb == pl.num_programs(1) - 1))
    def _finalize():
        total = (acc[0] / acc[1]
                 + acc[2] / (acc[3] * 4.0)
                 + acc[4] / (acc[3] * float(_HH)))
        out[...] = jnp.full((1, 1), total, dtype=jnp.float32)


def _fused_loss(pm5, cls2, logits, tdt, d2, tm6):
    return pl.pallas_call(
        _loss_body,
        grid=(_NB, _WB),
        in_specs=[
            pl.BlockSpec((_BR, 8, _C), lambda nb, wb: (nb, wb, 0)),
            pl.BlockSpec((_BN, 1), lambda nb, wb: (nb, 0)),
            pl.BlockSpec((_BN, _C), lambda nb, wb: (nb, 0)),
            pl.BlockSpec((_BN, 4 * _C), lambda nb, wb: (nb, 0)),
            pl.BlockSpec((_BN, 4 * _C), lambda nb, wb: (nb, 0)),
            pl.BlockSpec((_BN, 1, 1, _SEG), lambda nb, wb: (nb, wb, 0, 0)),
        ],
        out_specs=pl.BlockSpec((1, 1), lambda nb, wb: (0, 0)),
        out_shape=jax.ShapeDtypeStruct((1, 1), jnp.float32),
        scratch_shapes=[pltpu.SMEM((8,), jnp.float32)],
    )(pm5, cls2, logits, tdt, d2, tm6)


def kernel(target_deltas, mrcnn_deltas, mrcnn_class_logits, target_masks,
           pred_masks, target_class_ids):
    cls = target_class_ids.astype(jnp.int32)

    pm5 = pred_masks.reshape(_N * _H, _H, _C)

    tmp = jnp.pad(target_masks, ((0, 0), (0, 0), (0, 4)))
    tm6 = (tmp.reshape(_N, _H, _WB, 8).transpose(0, 2, 1, 3)
           .reshape(_N, _WB, 1, _SEG))

    d2 = mrcnn_deltas.reshape(_N, _C * 4)
    tdt = jnp.tile(target_deltas, (1, _C))

    out = _fused_loss(pm5, cls[:, None], mrcnn_class_logits, tdt, d2, tm6)
    return out[0, 0]


# speedup vs baseline: 2.0196x; 1.1821x over previous
import functools

import jax
import jax.numpy as jnp
from jax import lax
from jax.experimental import pallas as pl
from jax.experimental.pallas import tpu as pltpu

_N = 1024
_C = 81
_H = 28
_HH = _H * _H
_BN = 32
_NB = _N // _BN
_WB = 4
_BR = _BN * _H
_SEG = _H * 8


def _loss_body(pm_blk, cl, lg, td, d2, tm, out, acc):
    nb = pl.program_id(0)
    wb = pl.program_id(1)

    @pl.when((nb == 0) & (wb == 0))
    def _init():
        for k in range(5):
            acc[k] = 0.0

    cls = cl[...]
    safe = jnp.maximum(cls, 0)

    @pl.when(wb == 0)
    def _class_bbox():
        keep = (cls != -1).astype(jnp.float32)
        pos = (cls > 0).astype(jnp.float32)

        logits = lg[...]
        m = jnp.max(logits, axis=1, keepdims=True)
        lse = m + jnp.log(jnp.sum(jnp.exp(logits - m), axis=1,
                                  keepdims=True))
        iota = lax.broadcasted_iota(jnp.int32, logits.shape, 1)
        picked = jnp.sum(jnp.where(iota == safe, logits, 0.0), axis=1,
                         keepdims=True)
        acc[0] += jnp.sum(keep * (lse - picked))
        acc[1] += jnp.sum(keep)

        dv = d2[...]
        tdt = td[...]
        iota2 = lax.broadcasted_iota(jnp.int32, dv.shape, 1)
        selm = (iota2 // 4) == safe
        diff = jnp.abs(tdt - dv)
        lt = (diff < 1.0).astype(jnp.float32)
        sl1 = lt * 0.5 * diff * diff + (1.0 - lt) * (diff - 0.5)
        acc[2] += jnp.sum(jnp.where(selm, pos * sl1, 0.0))
        acc[3] += jnp.sum(pos)

    b = pm_blk[...].reshape(_BR * 8, _C)
    iota_c = lax.broadcasted_iota(jnp.int32, (_BN, _C), 1)
    onehot = (iota_c == safe).astype(jnp.float32)
    sel = lax.dot_general(onehot, b, (((1,), (1,)), ((), ())),
                          preferred_element_type=jnp.float32)

    valid = (lax.broadcasted_iota(jnp.int32, (1, _SEG), 1) % 8
             + 8 * wb) < _H
    pos_all = (cls > 0).astype(jnp.float32)
    s = 0.0
    for i in range(_BN):
        ypr = lax.slice(sel, (i, i * _SEG), (i + 1, (i + 1) * _SEG))
        tmn = tm[pl.ds(i, 1), 0, 0, :]
        posn = lax.slice(pos_all, (i, 0), (i + 1, 1))
        sce = (jnp.maximum(ypr, 0.0) - ypr * tmn
               + jnp.log1p(jnp.exp(-jnp.abs(ypr))))
        s = s + jnp.sum(posn * jnp.where(valid, sce, 0.0))
    acc[4] += s

    @pl.when((nb == pl.num_programs(0) - 1) & (wb == pl.num_programs(1) - 1))
    def _finalize():
        total = (acc[0] / acc[1]
                 + acc[2] / (acc[3] * 4.0)
                 + acc[4] / (acc[3] * float(_HH)))
        out[...] = jnp.full((1, 1), total, dtype=jnp.float32)


def _fused_loss(pm5, cls2, logits, tdt, d2, tm6):
    return pl.pallas_call(
        _loss_body,
        grid=(_NB, _WB),
        in_specs=[
            pl.BlockSpec((_BR, 8, _C), lambda nb, wb: (nb, wb, 0)),
            pl.BlockSpec((_BN, 1), lambda nb, wb: (nb, 0)),
            pl.BlockSpec((_BN, _C), lambda nb, wb: (nb, 0)),
            pl.BlockSpec((_BN, 4 * _C), lambda nb, wb: (nb, 0)),
            pl.BlockSpec((_BN, 4 * _C), lambda nb, wb: (nb, 0)),
            pl.BlockSpec((_BN, 1, 1, _SEG), lambda nb, wb: (nb, wb, 0, 0)),
        ],
        out_specs=pl.BlockSpec((1, 1), lambda nb, wb: (0, 0)),
        out_shape=jax.ShapeDtypeStruct((1, 1), jnp.float32),
        scratch_shapes=[pltpu.SMEM((8,), jnp.float32)],
    )(pm5, cls2, logits, tdt, d2, tm6)


def kernel(target_deltas, mrcnn_deltas, mrcnn_class_logits, target_masks,
           pred_masks, target_class_ids):
    cls = target_class_ids.astype(jnp.int32)

    pm5 = pred_masks.reshape(_N * _H, _H, _C)

    tmp = jnp.pad(target_masks, ((0, 0), (0, 0), (0, 4)))
    tm6 = (tmp.reshape(_N, _H, _WB, 8).transpose(0, 2, 1, 3)
           .reshape(_N, _WB, 1, _SEG))

    d2 = mrcnn_deltas.reshape(_N, _C * 4)
    tdt = jnp.tile(target_deltas, (1, _C))

    out = _fused_loss(pm5, cls[:, None], mrcnn_class_logits, tdt, d2, tm6)
    return out[0, 0]


# speedup vs baseline: 2.2422x; 1.1102x over previous
import functools

import jax
import jax.numpy as jnp
from jax import lax
from jax.experimental import pallas as pl
from jax.experimental.pallas import tpu as pltpu

_N = 1024
_C = 81
_H = 28
_HH = _H * _H
_BN = 64
_NB = _N // _BN
_WB = 4
_BR = _BN * _H
_SEG = _H * 8


def _loss_body(pm_blk, cl, lg, td, d2, tm, out, acc):
    nb = pl.program_id(0)
    wb = pl.program_id(1)

    @pl.when((nb == 0) & (wb == 0))
    def _init():
        for k in range(5):
            acc[k] = 0.0

    cls = cl[...]
    safe = jnp.maximum(cls, 0)

    @pl.when(wb == 0)
    def _class_bbox():
        keep = (cls != -1).astype(jnp.float32)
        pos = (cls > 0).astype(jnp.float32)

        logits = lg[...]
        m = jnp.max(logits, axis=1, keepdims=True)
        lse = m + jnp.log(jnp.sum(jnp.exp(logits - m), axis=1,
                                  keepdims=True))
        iota = lax.broadcasted_iota(jnp.int32, logits.shape, 1)
        picked = jnp.sum(jnp.where(iota == safe, logits, 0.0), axis=1,
                         keepdims=True)
        acc[0] += jnp.sum(keep * (lse - picked))
        acc[1] += jnp.sum(keep)

        dv = d2[...]
        tdt = td[...]
        iota2 = lax.broadcasted_iota(jnp.int32, dv.shape, 1)
        selm = (iota2 // 4) == safe
        diff = jnp.abs(tdt - dv)
        lt = (diff < 1.0).astype(jnp.float32)
        sl1 = lt * 0.5 * diff * diff + (1.0 - lt) * (diff - 0.5)
        acc[2] += jnp.sum(jnp.where(selm, pos * sl1, 0.0))
        acc[3] += jnp.sum(pos)

    b = pm_blk[...].reshape(_BR * 8, _C)
    iota_c = lax.broadcasted_iota(jnp.int32, (_BN, _C), 1)
    onehot = (iota_c == safe).astype(jnp.float32)
    sel = lax.dot_general(onehot, b, (((1,), (1,)), ((), ())),
                          preferred_element_type=jnp.float32)

    valid = (lax.broadcasted_iota(jnp.int32, (1, _SEG), 1) % 8
             + 8 * wb) < _H
    pos_all = (cls > 0).astype(jnp.float32)
    s = 0.0
    for i in range(_BN):
        ypr = lax.slice(sel, (i, i * _SEG), (i + 1, (i + 1) * _SEG))
        tmn = tm[pl.ds(i, 1), 0, 0, :]
        posn = lax.slice(pos_all, (i, 0), (i + 1, 1))
        sce = (jnp.maximum(ypr, 0.0) - ypr * tmn
               + jnp.log1p(jnp.exp(-jnp.abs(ypr))))
        s = s + jnp.sum(posn * jnp.where(valid, sce, 0.0))
    acc[4] += s

    @pl.when((nb == pl.num_programs(0) - 1) & (wb == pl.num_programs(1) - 1))
    def _finalize():
        total = (acc[0] / acc[1]
                 + acc[2] / (acc[3] * 4.0)
                 + acc[4] / (acc[3] * float(_HH)))
        out[...] = jnp.full((1, 1), total, dtype=jnp.float32)


def _fused_loss(pm5, cls2, logits, tdt, d2, tm6):
    return pl.pallas_call(
        _loss_body,
        grid=(_NB, _WB),
        in_specs=[
            pl.BlockSpec((_BR, 8, _C), lambda nb, wb: (nb, wb, 0)),
            pl.BlockSpec((_BN, 1), lambda nb, wb: (nb, 0)),
            pl.BlockSpec((_BN, _C), lambda nb, wb: (nb, 0)),
            pl.BlockSpec((_BN, 4 * _C), lambda nb, wb: (nb, 0)),
            pl.BlockSpec((_BN, 4 * _C), lambda nb, wb: (nb, 0)),
            pl.BlockSpec((_BN, 1, 1, _SEG), lambda nb, wb: (nb, wb, 0, 0)),
        ],
        out_specs=pl.BlockSpec((1, 1), lambda nb, wb: (0, 0)),
        out_shape=jax.ShapeDtypeStruct((1, 1), jnp.float32),
        scratch_shapes=[pltpu.SMEM((8,), jnp.float32)],
    )(pm5, cls2, logits, tdt, d2, tm6)


def kernel(target_deltas, mrcnn_deltas, mrcnn_class_logits, target_masks,
           pred_masks, target_class_ids):
    cls = target_class_ids.astype(jnp.int32)

    pm5 = pred_masks.reshape(_N * _H, _H, _C)

    tmp = jnp.pad(target_masks, ((0, 0), (0, 0), (0, 4)))
    tm6 = (tmp.reshape(_N, _H, _WB, 8).transpose(0, 2, 1, 3)
           .reshape(_N, _WB, 1, _SEG))

    d2 = mrcnn_deltas.reshape(_N, _C * 4)
    tdt = jnp.tile(target_deltas, (1, _C))

    out = _fused_loss(pm5, cls[:, None], mrcnn_class_logits, tdt, d2, tm6)
    return out[0, 0]


# speedup vs baseline: 2.3870x; 1.0646x over previous
import functools

import jax
import jax.numpy as jnp
from jax import lax
from jax.experimental import pallas as pl
from jax.experimental.pallas import tpu as pltpu

_N = 1024
_C = 81
_H = 28
_HH = _H * _H
_BN = 128
_NB = _N // _BN
_WB = 4
_BR = _BN * _H
_SEG = _H * 8


def _loss_body(pm_blk, cl, lg, td, d2, tm, out, acc):
    nb = pl.program_id(0)
    wb = pl.program_id(1)

    @pl.when((nb == 0) & (wb == 0))
    def _init():
        for k in range(5):
            acc[k] = 0.0

    cls = cl[...]
    safe = jnp.maximum(cls, 0)

    @pl.when(wb == 0)
    def _class_bbox():
        keep = (cls != -1).astype(jnp.float32)
        pos = (cls > 0).astype(jnp.float32)

        logits = lg[...]
        m = jnp.max(logits, axis=1, keepdims=True)
        lse = m + jnp.log(jnp.sum(jnp.exp(logits - m), axis=1,
                                  keepdims=True))
        iota = lax.broadcasted_iota(jnp.int32, logits.shape, 1)
        picked = jnp.sum(jnp.where(iota == safe, logits, 0.0), axis=1,
                         keepdims=True)
        acc[0] += jnp.sum(keep * (lse - picked))
        acc[1] += jnp.sum(keep)

        dv = d2[...]
        tdt = td[...]
        iota2 = lax.broadcasted_iota(jnp.int32, dv.shape, 1)
        selm = (iota2 // 4) == safe
        diff = jnp.abs(tdt - dv)
        lt = (diff < 1.0).astype(jnp.float32)
        sl1 = lt * 0.5 * diff * diff + (1.0 - lt) * (diff - 0.5)
        acc[2] += jnp.sum(jnp.where(selm, pos * sl1, 0.0))
        acc[3] += jnp.sum(pos)

    b = pm_blk[...].reshape(_BR * 8, _C)
    iota_c = lax.broadcasted_iota(jnp.int32, (_BN, _C), 1)
    onehot = (iota_c == safe).astype(jnp.float32)
    sel = lax.dot_general(onehot, b, (((1,), (1,)), ((), ())),
                          preferred_element_type=jnp.float32)

    valid = (lax.broadcasted_iota(jnp.int32, (1, _SEG), 1) % 8
             + 8 * wb) < _H
    pos_all = (cls > 0).astype(jnp.float32)
    s = 0.0
    for i in range(_BN):
        ypr = lax.slice(sel, (i, i * _SEG), (i + 1, (i + 1) * _SEG))
        tmn = tm[pl.ds(i, 1), 0, 0, :]
        posn = lax.slice(pos_all, (i, 0), (i + 1, 1))
        sce = (jnp.maximum(ypr, 0.0) - ypr * tmn
               + jnp.log1p(jnp.exp(-jnp.abs(ypr))))
        s = s + jnp.sum(posn * jnp.where(valid, sce, 0.0))
    acc[4] += s

    @pl.when((nb == pl.num_programs(0) - 1) & (wb == pl.num_programs(1) - 1))
    def _finalize():
        total = (acc[0] / acc[1]
                 + acc[2] / (acc[3] * 4.0)
                 + acc[4] / (acc[3] * float(_HH)))
        out[...] = jnp.full((1, 1), total, dtype=jnp.float32)


def _fused_loss(pm5, cls2, logits, tdt, d2, tm6):
    return pl.pallas_call(
        _loss_body,
        grid=(_NB, _WB),
        in_specs=[
            pl.BlockSpec((_BR, 8, _C), lambda nb, wb: (nb, wb, 0)),
            pl.BlockSpec((_BN, 1), lambda nb, wb: (nb, 0)),
            pl.BlockSpec((_BN, _C), lambda nb, wb: (nb, 0)),
            pl.BlockSpec((_BN, 4 * _C), lambda nb, wb: (nb, 0)),
            pl.BlockSpec((_BN, 4 * _C), lambda nb, wb: (nb, 0)),
            pl.BlockSpec((_BN, 1, 1, _SEG), lambda nb, wb: (nb, wb, 0, 0)),
        ],
        out_specs=pl.BlockSpec((1, 1), lambda nb, wb: (0, 0)),
        out_shape=jax.ShapeDtypeStruct((1, 1), jnp.float32),
        scratch_shapes=[pltpu.SMEM((8,), jnp.float32)],
    )(pm5, cls2, logits, tdt, d2, tm6)


def kernel(target_deltas, mrcnn_deltas, mrcnn_class_logits, target_masks,
           pred_masks, target_class_ids):
    cls = target_class_ids.astype(jnp.int32)

    pm5 = pred_masks.reshape(_N * _H, _H, _C)

    tmp = jnp.pad(target_masks, ((0, 0), (0, 0), (0, 4)))
    tm6 = (tmp.reshape(_N, _H, _WB, 8).transpose(0, 2, 1, 3)
           .reshape(_N, _WB, 1, _SEG))

    d2 = mrcnn_deltas.reshape(_N, _C * 4)
    tdt = jnp.tile(target_deltas, (1, _C))

    out = _fused_loss(pm5, cls[:, None], mrcnn_class_logits, tdt, d2, tm6)
    return out[0, 0]


# speedup vs baseline: 2.3885x; 1.0007x over previous
import functools

import jax
import jax.numpy as jnp
from jax import lax
from jax.experimental import pallas as pl
from jax.experimental.pallas import tpu as pltpu

_N = 1024
_C = 81
_H = 28
_HH = _H * _H
_BN = 128
_NB = _N // _BN
_WB = 4
_BR = _BN * _H
_SEG = _H * 8


def _loss_body(pm_blk, cl, lg, td, d2, tm, out, acc):
    nb = pl.program_id(0)
    wb = pl.program_id(1)

    @pl.when((nb == 0) & (wb == 0))
    def _init():
        for k in range(5):
            acc[k] = 0.0

    cls = cl[...]
    safe = jnp.maximum(cls, 0)

    @pl.when(wb == 0)
    def _class_bbox():
        keep = (cls != -1).astype(jnp.float32)
        pos = (cls > 0).astype(jnp.float32)

        logits = lg[...]
        m = jnp.max(logits, axis=1, keepdims=True)
        lse = m + jnp.log(jnp.sum(jnp.exp(logits - m), axis=1,
                                  keepdims=True))
        iota = lax.broadcasted_iota(jnp.int32, logits.shape, 1)
        picked = jnp.sum(jnp.where(iota == safe, logits, 0.0), axis=1,
                         keepdims=True)
        acc[0] += jnp.sum(keep * (lse - picked))
        acc[1] += jnp.sum(keep)

        dv = d2[...]
        tdt = td[...]
        iota2 = lax.broadcasted_iota(jnp.int32, dv.shape, 1)
        selm = (iota2 // 4) == safe
        diff = jnp.abs(tdt - dv)
        lt = (diff < 1.0).astype(jnp.float32)
        sl1 = lt * 0.5 * diff * diff + (1.0 - lt) * (diff - 0.5)
        acc[2] += jnp.sum(jnp.where(selm, pos * sl1, 0.0))
        acc[3] += jnp.sum(pos)

    b = pm_blk[...].reshape(_BR * 8, _C)
    iota_c = lax.broadcasted_iota(jnp.int32, (_BN, _C), 1)
    onehot = (iota_c == safe).astype(jnp.float32)

    valid = (lax.broadcasted_iota(jnp.int32, (1, _SEG), 1) % 8
             + 8 * wb) < _H
    pos_all = (cls > 0).astype(jnp.float32)
    s = 0.0
    for q in range(_BN // 16):
        oh_q = lax.slice(onehot, (q * 16, 0), ((q + 1) * 16, _C))
        b_q = lax.slice(b, (q * 16 * _SEG, 0), ((q + 1) * 16 * _SEG, _C))
        sel = lax.dot_general(oh_q, b_q, (((1,), (1,)), ((), ())),
                              preferred_element_type=jnp.float32)
        for p in range(16):
            i = q * 16 + p
            ypr = lax.slice(sel, (p, p * _SEG), (p + 1, (p + 1) * _SEG))
            tmn = tm[pl.ds(i, 1), 0, 0, :]
            posn = lax.slice(pos_all, (i, 0), (i + 1, 1))
            sce = (jnp.maximum(ypr, 0.0) - ypr * tmn
                   + jnp.log1p(jnp.exp(-jnp.abs(ypr))))
            s = s + jnp.sum(posn * jnp.where(valid, sce, 0.0))
    acc[4] += s

    @pl.when((nb == pl.num_programs(0) - 1) & (wb == pl.num_programs(1) - 1))
    def _finalize():
        total = (acc[0] / acc[1]
                 + acc[2] / (acc[3] * 4.0)
                 + acc[4] / (acc[3] * float(_HH)))
        out[...] = jnp.full((1, 1), total, dtype=jnp.float32)


def _fused_loss(pm5, cls2, logits, tdt, d2, tm6):
    return pl.pallas_call(
        _loss_body,
        grid=(_NB, _WB),
        in_specs=[
            pl.BlockSpec((_BR, 8, _C), lambda nb, wb: (nb, wb, 0)),
            pl.BlockSpec((_BN, 1), lambda nb, wb: (nb, 0)),
            pl.BlockSpec((_BN, _C), lambda nb, wb: (nb, 0)),
            pl.BlockSpec((_BN, 4 * _C), lambda nb, wb: (nb, 0)),
            pl.BlockSpec((_BN, 4 * _C), lambda nb, wb: (nb, 0)),
            pl.BlockSpec((_BN, 1, 1, _SEG), lambda nb, wb: (nb, wb, 0, 0)),
        ],
        out_specs=pl.BlockSpec((1, 1), lambda nb, wb: (0, 0)),
        out_shape=jax.ShapeDtypeStruct((1, 1), jnp.float32),
        scratch_shapes=[pltpu.SMEM((8,), jnp.float32)],
    )(pm5, cls2, logits, tdt, d2, tm6)


def kernel(target_deltas, mrcnn_deltas, mrcnn_class_logits, target_masks,
           pred_masks, target_class_ids):
    cls = target_class_ids.astype(jnp.int32)

    pm5 = pred_masks.reshape(_N * _H, _H, _C)

    tmp = jnp.pad(target_masks, ((0, 0), (0, 0), (0, 4)))
    tm6 = (tmp.reshape(_N, _H, _WB, 8).transpose(0, 2, 1, 3)
           .reshape(_N, _WB, 1, _SEG))

    d2 = mrcnn_deltas.reshape(_N, _C * 4)
    tdt = jnp.tile(target_deltas, (1, _C))

    out = _fused_loss(pm5, cls[:, None], mrcnn_class_logits, tdt, d2, tm6)
    return out[0, 0]
